# Initial kernel scaffold; baseline (speedup 1.0000x reference)
import jax
import jax.numpy as jnp
from jax.experimental import pallas as pl


def kernel(position_ids, table):
    raise NotImplementedError("write your pallas kernel here")


# speedup vs baseline: 2.1567x; 2.1567x over previous
import functools

import jax
import jax.numpy as jnp
from jax import lax
from jax.experimental import pallas as pl
from jax.experimental.pallas import tpu as pltpu
from jax.experimental.pallas import tpu_sc as plsc

NUM_POSITIONS = 1024
HIDDEN = 768
BATCH = 64
SEQ = 1024

NC = 2
NS = 16
NW = NC * NS

TOTAL = BATCH * SEQ
BPW = TOTAL // NW
CHUNK = 64
NCHUNK = BPW // CHUNK

_mesh = plsc.VectorSubcoreMesh(core_axis_name="c", subcore_axis_name="s")


@functools.partial(
    pl.kernel,
    mesh=_mesh,
    out_type=jax.ShapeDtypeStruct((TOTAL, HIDDEN), jnp.float32),
    scratch_types=[
        pltpu.VMEM((NCHUNK, CHUNK), jnp.int32),
        pltpu.VMEM((CHUNK, HIDDEN), jnp.float32),
        pltpu.SemaphoreType.DMA,
    ],
)
def _gather_rows(ids_hbm, table_hbm, out_hbm, idx_v, rows_v, gsem):
    wid = lax.axis_index("s") * NC + lax.axis_index("c")
    base = wid * BPW
    pltpu.sync_copy(ids_hbm.at[wid], idx_v)

    def step(ci, carry):
        pltpu.async_copy(table_hbm.at[idx_v.at[ci]], rows_v, gsem).wait()
        pltpu.sync_copy(rows_v, out_hbm.at[pl.ds(base + ci * CHUNK, CHUNK)])
        return carry

    lax.fori_loop(0, NCHUNK, step, 0)


def kernel(position_ids, table):
    ids = jnp.reshape(position_ids.astype(jnp.int32), (NW, NCHUNK, CHUNK))
    out = _gather_rows(ids, table)
    return jnp.reshape(out, (BATCH, SEQ, HIDDEN))


# speedup vs baseline: 2.3111x; 1.0716x over previous
import functools

import jax
import jax.numpy as jnp
from jax import lax
from jax.experimental import pallas as pl
from jax.experimental.pallas import tpu as pltpu
from jax.experimental.pallas import tpu_sc as plsc

NUM_POSITIONS = 1024
HIDDEN = 768
BATCH = 64
SEQ = 1024

NC = 2
NS = 16
NW = NC * NS

TOTAL = BATCH * SEQ
BPW = TOTAL // NW
CHUNK = 64
NCHUNK = BPW // CHUNK

_mesh = plsc.VectorSubcoreMesh(core_axis_name="c", subcore_axis_name="s")


@functools.partial(
    pl.kernel,
    mesh=_mesh,
    out_type=jax.ShapeDtypeStruct((TOTAL, HIDDEN), jnp.float32),
    scratch_types=[
        pltpu.VMEM((NCHUNK, CHUNK), jnp.int32),
        pltpu.VMEM((2, CHUNK, HIDDEN), jnp.float32),
        pltpu.SemaphoreType.DMA,
        pltpu.SemaphoreType.DMA,
        pltpu.SemaphoreType.DMA,
        pltpu.SemaphoreType.DMA,
    ],
)
def _gather_rows(ids_hbm, table_hbm, out_hbm, idx_v, rows_v, g0, g1, w0, w1):
    wid = lax.axis_index("s") * NC + lax.axis_index("c")
    base = wid * BPW
    gs = (g0, g1)
    ws = (w0, w1)
    pltpu.sync_copy(ids_hbm.at[wid], idx_v)

    def start_gather(ci, b):
        pltpu.async_copy(table_hbm.at[idx_v.at[ci]], rows_v.at[b], gs[b])

    def wait_gather(b):
        pltpu.make_async_copy(
            table_hbm.at[idx_v.at[0]], rows_v.at[b], gs[b]).wait()

    def start_write(ci, b):
        pltpu.async_copy(
            rows_v.at[b], out_hbm.at[pl.ds(base + ci * CHUNK, CHUNK)], ws[b])

    def wait_write(b):
        pltpu.make_async_copy(
            rows_v.at[b], out_hbm.at[pl.ds(base, CHUNK)], ws[b]).wait()

    def step(o, carry):
        first = o > 0

        @pl.when(first)
        def _():
            wait_write(0)

        start_gather(2 * o, 0)

        @pl.when(first)
        def _():
            wait_gather(1)
            start_write(2 * o - 1, 1)

        @pl.when(first)
        def _():
            wait_write(1)

        start_gather(2 * o + 1, 1)
        wait_gather(0)
        start_write(2 * o, 0)
        return carry

    lax.fori_loop(0, NCHUNK // 2, step, 0)
    wait_gather(1)
    start_write(NCHUNK - 1, 1)
    wait_write(0)
    wait_write(1)


def kernel(position_ids, table):
    ids = jnp.reshape(position_ids.astype(jnp.int32), (NW, NCHUNK, CHUNK))
    out = _gather_rows(ids, table)
    return jnp.reshape(out, (BATCH, SEQ, HIDDEN))
